# baseline (device time: 13726 ns/iter reference)
import jax
import jax.numpy as jnp
from jax import lax
from jax.experimental import pallas as pl
from jax.experimental.pallas import tpu as pltpu

N_DEV = 4


def kernel(x, w_mat):
    m_per, k = x.shape
    n = w_mat.shape[1]
    n_per = n // N_DEV

    def body(x_hbm, w_hbm, out_ref, xv, wv, stage, rbuf,
             copy_sems, send_sems, recv_sems):
        my = lax.axis_index("i")

        cp_x = pltpu.make_async_copy(x_hbm, xv, copy_sems.at[0])
        cp_x.start()
        w_cps = []
        for j in range(N_DEV):
            cp = pltpu.make_async_copy(
                w_hbm.at[:, pl.ds(j * n_per, n_per)], wv.at[j],
                copy_sems.at[1 + j])
            cp.start()
            w_cps.append(cp)

        barrier_sem = pltpu.get_barrier_semaphore()
        for d in range(N_DEV):
            @pl.when(my != d)
            def _():
                pl.semaphore_signal(
                    barrier_sem, inc=1,
                    device_id=(d,), device_id_type=pl.DeviceIdType.MESH,
                )
        pl.semaphore_wait(barrier_sem, N_DEV - 1)

        cp_x.wait()
        xb = xv[...].astype(jnp.bfloat16)

        def send_desc(j):
            return pltpu.make_async_remote_copy(
                src_ref=stage.at[j],
                dst_ref=rbuf.at[my],
                send_sem=send_sems.at[j],
                recv_sem=recv_sems.at[my],
                device_id=(j,),
                device_id_type=pl.DeviceIdType.MESH,
            )

        def recv_desc(s):
            return pltpu.make_async_remote_copy(
                src_ref=stage.at[s],
                dst_ref=rbuf.at[s],
                send_sem=send_sems.at[s],
                recv_sem=recv_sems.at[s],
                device_id=(s,),
                device_id_type=pl.DeviceIdType.MESH,
            )

        for j in range(N_DEV):
            w_cps[j].wait()
            y = jnp.dot(xb, wv[j].astype(jnp.bfloat16),
                        preferred_element_type=jnp.float32)
            y = y * jax.nn.sigmoid(y)

            @pl.when(my == j)
            def _():
                out_ref[pl.ds(my * m_per, m_per), :] = y

            @pl.when(my != j)
            def _():
                stage[j] = y.astype(jnp.bfloat16)
                send_desc(j).start()

        for s in range(N_DEV):
            @pl.when(my != s)
            def _():
                recv_desc(s).wait_recv()
                out_ref[pl.ds(s * m_per, m_per), :] = rbuf[s].astype(jnp.float32)

        for j in range(N_DEV):
            @pl.when(my != j)
            def _():
                send_desc(j).wait_send()

    return pl.pallas_call(
        body,
        out_shape=jax.ShapeDtypeStruct((N_DEV * m_per, n_per), jnp.float32),
        in_specs=[
            pl.BlockSpec(memory_space=pl.ANY),
            pl.BlockSpec(memory_space=pl.ANY),
        ],
        out_specs=pl.BlockSpec(memory_space=pltpu.VMEM),
        scratch_shapes=[
            pltpu.VMEM((m_per, k), jnp.float32),
            pltpu.VMEM((N_DEV, k, n_per), jnp.float32),
            pltpu.VMEM((N_DEV, m_per, n_per), jnp.bfloat16),
            pltpu.VMEM((N_DEV, m_per, n_per), jnp.bfloat16),
            pltpu.SemaphoreType.DMA((N_DEV + 1,)),
            pltpu.SemaphoreType.DMA((N_DEV,)),
            pltpu.SemaphoreType.DMA((N_DEV,)),
        ],
        compiler_params=pltpu.CompilerParams(collective_id=0),
    )(x, w_mat)


# device time: 12415 ns/iter; 1.1056x vs baseline; 1.1056x over previous
import jax
import jax.numpy as jnp
from jax import lax
from jax.experimental import pallas as pl
from jax.experimental.pallas import tpu as pltpu

N_DEV = 4


def kernel(x, w_mat):
    m_per, k = x.shape
    n = w_mat.shape[1]
    n_per = n // N_DEV

    def body(x_ref, w_ref, out_ref, own, stage, rbuf, send_sems, recv_sems):
        my = lax.axis_index("i")

        barrier_sem = pltpu.get_barrier_semaphore()
        for d in range(N_DEV):
            @pl.when(my != d)
            def _():
                pl.semaphore_signal(
                    barrier_sem, inc=1,
                    device_id=(d,), device_id_type=pl.DeviceIdType.MESH,
                )

        xb = x_ref[...].astype(jnp.bfloat16)

        def send_desc(j):
            return pltpu.make_async_remote_copy(
                src_ref=stage.at[j],
                dst_ref=rbuf.at[my],
                send_sem=send_sems.at[j],
                recv_sem=recv_sems.at[my],
                device_id=(j,),
                device_id_type=pl.DeviceIdType.MESH,
            )

        def recv_desc(s):
            return pltpu.make_async_remote_copy(
                src_ref=stage.at[s],
                dst_ref=rbuf.at[s],
                send_sem=send_sems.at[s],
                recv_sem=recv_sems.at[s],
                device_id=(s,),
                device_id_type=pl.DeviceIdType.MESH,
            )

        barrier_done = False
        for j in range(N_DEV):
            y = jnp.dot(xb, w_ref[:, j * n_per:(j + 1) * n_per].astype(jnp.bfloat16),
                        preferred_element_type=jnp.float32)
            y = y * jax.nn.sigmoid(y)

            @pl.when(my == j)
            def _():
                own[...] = y

            if not barrier_done:
                pl.semaphore_wait(barrier_sem, N_DEV - 1)
                barrier_done = True

            @pl.when(my != j)
            def _():
                stage[j] = y.astype(jnp.bfloat16)
                send_desc(j).start()

        out_ref[pl.ds(my * m_per, m_per), :] = own[...]

        for s in range(N_DEV):
            @pl.when(my != s)
            def _():
                recv_desc(s).wait_recv()
                out_ref[pl.ds(s * m_per, m_per), :] = rbuf[s].astype(jnp.float32)

        for j in range(N_DEV):
            @pl.when(my != j)
            def _():
                send_desc(j).wait_send()

    return pl.pallas_call(
        body,
        out_shape=jax.ShapeDtypeStruct((N_DEV * m_per, n_per), jnp.float32),
        in_specs=[
            pl.BlockSpec(memory_space=pltpu.VMEM),
            pl.BlockSpec(memory_space=pltpu.VMEM),
        ],
        out_specs=pl.BlockSpec(memory_space=pltpu.VMEM),
        scratch_shapes=[
            pltpu.VMEM((m_per, n_per), jnp.float32),
            pltpu.VMEM((N_DEV, m_per, n_per), jnp.bfloat16),
            pltpu.VMEM((N_DEV, m_per, n_per), jnp.bfloat16),
            pltpu.SemaphoreType.DMA((N_DEV,)),
            pltpu.SemaphoreType.DMA((N_DEV,)),
        ],
        compiler_params=pltpu.CompilerParams(collective_id=0),
    )(x, w_mat)
